# Initial kernel scaffold; baseline (speedup 1.0000x reference)
import jax
import jax.numpy as jnp
from jax.experimental import pallas as pl


def kernel(node_features, edge_index, edge_weight, batch, W_in, b_in, lin_w, att_src, att_dst, lin_edge_w, att_edge, gat_bias, ln_g, ln_b, gate_w1, gate_b1, gate_w2, gate_b2, mlp_w1, mlp_b1, mlp_w2, mlp_b2, mlp_w3, mlp_b3):
    raise NotImplementedError("write your pallas kernel here")


# speedup vs baseline: 41.4950x; 41.4950x over previous
import functools

import jax
import jax.numpy as jnp
from jax import lax
from jax.experimental import pallas as pl
from jax.experimental.pallas import tpu as pltpu
from jax.experimental.pallas import tpu_sc as plsc

F32 = jnp.float32
I32 = jnp.int32

NBLK = 1000
K = 80
NSUB = 16
NCORE = 2
NW = NSUB * NCORE


def _edge_stats_body(ew_ref, out_ref):
    ew = ew_ref[...]
    mn = jnp.min(ew)
    mx = jnp.max(ew)
    sm = jnp.sum(ew)
    lane = lax.broadcasted_iota(I32, (1, 128), 1)
    out_ref[...] = jnp.where(lane == 0, mn,
                             jnp.where(lane == 1, mx,
                                       jnp.where(lane == 2, sm, 0.0)))


def _in_proj_body(nf_ref, w_ref, b_ref, out_ref):
    out_ref[...] = lax.dot_general(
        nf_ref[...], w_ref[...], (((1,), (1,)), ((), ())),
        preferred_element_type=F32,
        precision=lax.Precision.HIGHEST) + b_ref[...]


def _proj_body(x_ref, lw_ref, aa_ref, xp_ref, asd_ref, mx_ref):
    i = pl.program_id(0)
    x = x_ref[...]
    xp_ref[...] = lax.dot_general(x, lw_ref[...], (((1,), (1,)), ((), ())),
                                  preferred_element_type=F32,
        precision=lax.Precision.HIGHEST)
    asd = lax.dot_general(x, aa_ref[...], (((1,), (0,)), ((), ())),
                          preferred_element_type=F32,
        precision=lax.Precision.HIGHEST)
    asd_ref[...] = asd
    bm = jnp.max(asd, axis=0, keepdims=True)

    @pl.when(i == 0)
    def _():
        mx_ref[...] = bm

    @pl.when(i > 0)
    def _():
        mx_ref[...] = jnp.maximum(mx_ref[...], bm)


def _combine_body(acc_ref, asd_ref, xp_ref, xres_ref, bias_ref, lg_ref,
                  lb_ref, consts_ref, out_ref, *, with_res):
    acc = acc_ref[...]
    a0 = acc[0]
    a1 = acc[1]
    out_raw = a0[:, :128] + a1[:, :128]
    den4 = a0[:, 128:132] + a1[:, 128:132]
    asd = asd_ref[...]
    a_s = asd[:, :4]
    a_d = asd[:, 4:8]
    cst = consts_ref[...]
    srow = cst[:, 0:4]
    cerow = cst[:, 4:8]
    fw = cst[0, 8]
    al = a_s + a_d + fw * cerow
    al = jnp.where(al >= 0, al, al * 0.2)
    exsl = jnp.exp(al - srow)
    den4 = den4 + exsl
    hh = lax.broadcasted_iota(I32, (4, 128), 0)
    ll = lax.broadcasted_iota(I32, (4, 128), 1) // 32
    rmat = jnp.where(hh == ll, 1.0, 0.0).astype(F32)
    den_w = lax.dot_general(den4, rmat, (((1,), (0,)), ((), ())),
                            preferred_element_type=F32,
        precision=lax.Precision.HIGHEST)
    exsl_w = lax.dot_general(exsl, rmat, (((1,), (0,)), ((), ())),
                             preferred_element_type=F32,
        precision=lax.Precision.HIGHEST)
    xp = xp_ref[...]
    out = (out_raw + exsl_w * xp) / (den_w + 1e-16) + bias_ref[...]
    h = jnp.where(out > 0, out, jnp.exp(jnp.minimum(out, 0.0)) - 1.0)
    m = jnp.mean(h, axis=1, keepdims=True)
    v = jnp.mean((h - m) ** 2, axis=1, keepdims=True)
    hn = (h - m) / jnp.sqrt(v + 1e-5) * lg_ref[...] + lb_ref[...]
    if with_res:
        hn = hn + xres_ref[...]
    out_ref[...] = hn


def _gate_body(x_ref, g1_ref, b1_ref, g2_ref, b2_ref, gout_ref, stats_ref,
               m_scr, z_scr):
    i = pl.program_id(0)
    nb = pl.num_programs(0)
    t = jnp.tanh(lax.dot_general(x_ref[...], g1_ref[...],
                                 (((1,), (1,)), ((), ())),
                                 preferred_element_type=F32,
        precision=lax.Precision.HIGHEST) + b1_ref[...])
    gb = lax.dot_general(g2_ref[...], t, (((1,), (1,)), ((), ())),
                         preferred_element_type=F32,
        precision=lax.Precision.HIGHEST) + b2_ref[0, 0]
    gout_ref[...] = gb[None]
    bm = jnp.max(gb)

    @pl.when(i == 0)
    def _():
        m_scr[0] = bm
        z_scr[0] = jnp.sum(jnp.exp(gb - bm))

    @pl.when(i > 0)
    def _():
        m_old = m_scr[0]
        m_new = jnp.maximum(m_old, bm)
        z_scr[0] = z_scr[0] * jnp.exp(m_old - m_new) + \
            jnp.sum(jnp.exp(gb - m_new))
        m_scr[0] = m_new

    @pl.when(i == nb - 1)
    def _():
        lane = lax.broadcasted_iota(I32, (1, 2), 1)
        stats_ref[...] = jnp.where(lane == 0, m_scr[0], z_scr[0])


def _pool_body(x_ref, g_ref, batch_ref, stats_ref, w1_ref, b1_ref, w2_ref,
               b2_ref, w3_ref, b3_ref, out_ref, pool_scr):
    i = pl.program_id(0)
    nb = pl.num_programs(0)
    blk = x_ref.shape[0]
    st = stats_ref[...]
    m = st[0, 0]
    z = st[0, 1]
    gb = g_ref[...][0]
    wgt = jnp.exp(gb - m) / z
    bb = batch_ref[...][0]
    rows = lax.broadcasted_iota(I32, (64, blk), 0)
    oh = jnp.where(rows == jnp.broadcast_to(bb, (64, blk)), 1.0, 0.0)
    ohw = oh.astype(F32) * wgt
    con = lax.dot_general(ohw, x_ref[...], (((1,), (0,)), ((), ())),
                          preferred_element_type=F32,
        precision=lax.Precision.HIGHEST)

    @pl.when(i == 0)
    def _():
        pool_scr[...] = con

    @pl.when(i > 0)
    def _():
        pool_scr[...] = pool_scr[...] + con

    @pl.when(i == nb - 1)
    def _():
        p = pool_scr[...]
        h1 = lax.dot_general(p, w1_ref[...], (((1,), (1,)), ((), ())),
                             preferred_element_type=F32,
        precision=lax.Precision.HIGHEST) + b1_ref[...]
        h1 = jnp.maximum(h1, 0.0)
        h2 = lax.dot_general(h1, w2_ref[...], (((1,), (1,)), ((), ())),
                             preferred_element_type=F32,
        precision=lax.Precision.HIGHEST) + b2_ref[...]
        h2 = jnp.maximum(h2, 0.0)
        o = lax.dot_general(w3_ref[...], h2, (((1,), (1,)), ((), ())),
                            preferred_element_type=F32,
        precision=lax.Precision.HIGHEST) + b3_ref[0, 0]
        out_ref[...] = o


def _sc_edge_body(src_hbm, dst_hbm, w_hbm, asd_hbm, xp_hbm, consts_hbm,
                  acc_hbm, consts_v, src_v, dst_v, w_v, asrows_v, adrows_v,
                  rows_v, rows144_v, acc_sp, sem):
    e = src_hbm.shape[0]
    npad = acc_hbm.shape[1]
    c = lax.axis_index("c")
    s = lax.axis_index("s")
    wid = s * NCORE + c
    epw = e // NW
    nch = epw // K
    rpt = npad // NSUB

    pltpu.sync_copy(consts_hbm, consts_v)
    cv = consts_v[...]
    iota16 = jnp.arange(16, dtype=I32)

    z16 = jnp.zeros((16,), F32)
    cols = [(iota16 + 16 * j) for j in range(9)]

    def zrow(r, carry):
        rs = jnp.full((16,), r, I32)
        for j in range(9):
            plsc.store_scatter(rows144_v, [rs, cols[j]], z16)
        return carry

    lax.fori_loop(0, K, zrow, 0)

    r0 = s * rpt

    def zchunk(kk, carry):
        pltpu.sync_copy(rows144_v, acc_sp.at[pl.ds(r0 + kk * K, K)])
        return carry

    lax.fori_loop(0, rpt // K, zchunk, 0)
    plsc.subcore_barrier()

    ccols = [(iota16 + 32 * h + 16 * half)
             for h in range(4) for half in range(2)]
    ecols = [jnp.full((16,), 128 + h, I32) for h in range(4)]
    hrows = [jnp.full((16,), h, I32) for h in range(8)]

    def chunk(ch, carry):
        base = wid * epw + ch * K
        pltpu.sync_copy(src_hbm.at[pl.ds(base, K)], src_v)
        pltpu.sync_copy(dst_hbm.at[pl.ds(base, K)], dst_v)
        pltpu.sync_copy(w_hbm.at[pl.ds(base, K)], w_v)
        pltpu.async_copy(asd_hbm.at[src_v], asrows_v, sem).wait()
        pltpu.async_copy(asd_hbm.at[dst_v], adrows_v, sem).wait()
        pltpu.async_copy(xp_hbm.at[src_v], rows_v, sem).wait()

        def group(g, gcarry):
            gi = g * 16 + iota16
            sg = plsc.load_gather(src_v, [gi])
            dg = plsc.load_gather(dst_v, [gi])
            wg = plsc.load_gather(w_v, [gi])
            exs = []
            for h in range(4):
                a_s = plsc.load_gather(asrows_v, [gi, hrows[h]])
                a_d = plsc.load_gather(adrows_v, [gi, hrows[4 + h]])
                al = a_s + a_d + wg * cv[4 + h]
                al = jnp.where(al >= 0, al, al * 0.2)
                ex = jnp.exp(al - cv[h])
                exs.append(ex)
                plsc.store_scatter(rows144_v, [gi, ecols[h]], ex)
            for ei in range(16):
                rs = jnp.full((16,), g * 16 + ei, I32)
                for h in range(4):
                    sc = exs[h][ei]
                    for half in range(2):
                        cc = ccols[h * 2 + half]
                        lg = plsc.load_gather(rows_v, [rs, cc])
                        plsc.store_scatter(rows144_v, [rs, cc], lg * sc)
            return gcarry

        lax.fori_loop(0, K // 16, group, 0)
        pltpu.sync_copy(rows144_v, acc_sp.at[dst_v], add=True)
        return carry

    lax.fori_loop(0, nch, chunk, 0)
    plsc.subcore_barrier()
    pltpu.sync_copy(acc_sp.at[pl.ds(r0, rpt)],
                    acc_hbm.at[c, pl.ds(r0, rpt)])


def kernel(node_features, edge_index, edge_weight, batch, W_in, b_in, lin_w,
           att_src, att_dst, lin_edge_w, att_edge, gat_bias, ln_g, ln_b,
           gate_w1, gate_b1, gate_w2, gate_b2, mlp_w1, mlp_b1, mlp_w2,
           mlp_b2, mlp_w3, mlp_b3):
    n, _ = node_features.shape
    e = edge_weight.shape[0]
    num_layers, hid, _ = lin_w.shape
    nheads = att_src.shape[1]
    nb = n // NBLK
    src = edge_index[0]
    dst = edge_index[1]

    lw4 = lin_w.reshape(num_layers, nheads, hid // nheads, hid)
    fold_s = jnp.einsum('lhck,lhc->lkh', lw4, att_src)
    fold_d = jnp.einsum('lhck,lhc->lkh', lw4, att_dst)
    asad = jnp.concatenate([fold_s, fold_d], axis=2)
    ce = jnp.einsum('lhc,lhc->lh',
                    lin_edge_w.reshape(num_layers, nheads, hid // nheads),
                    att_edge)

    stats = pl.pallas_call(
        _edge_stats_body,
        grid=(1,),
        in_specs=[pl.BlockSpec((e // 128, 128), lambda i: (0, 0))],
        out_specs=pl.BlockSpec((1, 128), lambda i: (0, 0)),
        out_shape=jax.ShapeDtypeStruct((1, 128), F32),
    )(edge_weight.reshape(e // 128, 128))
    minw = stats[0, 0]
    maxw = stats[0, 1]
    fw = stats[0, 2] / e

    x = pl.pallas_call(
        _in_proj_body,
        grid=(nb,),
        in_specs=[
            pl.BlockSpec((NBLK, 128), lambda i: (i, 0)),
            pl.BlockSpec((128, 128), lambda i: (0, 0)),
            pl.BlockSpec((1, 128), lambda i: (0, 0)),
        ],
        out_specs=pl.BlockSpec((NBLK, 128), lambda i: (i, 0)),
        out_shape=jax.ShapeDtypeStruct((n, hid), F32),
    )(node_features, W_in, b_in.reshape(1, hid))

    sc_call = pl.kernel(
        _sc_edge_body,
        out_type=jax.ShapeDtypeStruct((2, 10240, 144), F32),
        mesh=plsc.VectorSubcoreMesh(core_axis_name="c", subcore_axis_name="s"),
        compiler_params=pltpu.CompilerParams(use_tc_tiling_on_sc=False,
                                             needs_layout_passes=False),
        scratch_types=[
            pltpu.VMEM((16,), F32),
            pltpu.VMEM((K,), I32),
            pltpu.VMEM((K,), I32),
            pltpu.VMEM((K,), F32),
            pltpu.VMEM((K, 8), F32),
            pltpu.VMEM((K, 8), F32),
            pltpu.VMEM((K, 128), F32),
            pltpu.VMEM((K, 144), F32),
            pltpu.VMEM_SHARED((10240, 144), F32),
            pltpu.SemaphoreType.DMA,
        ],
    )

    for i in range(num_layers):
        xp, asd, mx = pl.pallas_call(
            functools.partial(_proj_body),
            grid=(nb,),
            in_specs=[
                pl.BlockSpec((NBLK, 128), lambda i: (i, 0)),
                pl.BlockSpec((128, 128), lambda i: (0, 0)),
                pl.BlockSpec((128, 8), lambda i: (0, 0)),
            ],
            out_specs=[
                pl.BlockSpec((NBLK, 128), lambda i: (i, 0)),
                pl.BlockSpec((NBLK, 8), lambda i: (i, 0)),
                pl.BlockSpec((1, 8), lambda i: (0, 0)),
            ],
            out_shape=[
                jax.ShapeDtypeStruct((n, hid), F32),
                jax.ShapeDtypeStruct((n, 8), F32),
                jax.ShapeDtypeStruct((1, 8), F32),
            ],
        )(x, lin_w[i], asad[i])

        cei = ce[i]
        bound = mx[0, :4] + mx[0, 4:8] + jnp.maximum(
            jnp.maximum(cei * minw, cei * maxw), cei * fw)
        s = jnp.where(bound >= 0, bound, 0.2 * bound)
        consts = jnp.concatenate(
            [s, cei, jnp.stack([fw]), jnp.zeros((7,), F32)]).astype(F32)

        acc = sc_call(src, dst, edge_weight, asd, xp, consts)

        x = pl.pallas_call(
            functools.partial(_combine_body, with_res=(i > 0)),
            grid=(nb,),
            in_specs=[
                pl.BlockSpec((2, NBLK, 144), lambda i: (0, i, 0)),
                pl.BlockSpec((NBLK, 8), lambda i: (i, 0)),
                pl.BlockSpec((NBLK, 128), lambda i: (i, 0)),
                pl.BlockSpec((NBLK, 128), lambda i: (i, 0)),
                pl.BlockSpec((1, 128), lambda i: (0, 0)),
                pl.BlockSpec((1, 128), lambda i: (0, 0)),
                pl.BlockSpec((1, 128), lambda i: (0, 0)),
                pl.BlockSpec((1, 16), lambda i: (0, 0)),
            ],
            out_specs=pl.BlockSpec((NBLK, 128), lambda i: (i, 0)),
            out_shape=jax.ShapeDtypeStruct((n, hid), F32),
        )(acc, asd, xp, x, gat_bias[i].reshape(1, hid),
          ln_g[i].reshape(1, hid), ln_b[i].reshape(1, hid),
          consts.reshape(1, 16))

    g3, gstats = pl.pallas_call(
        _gate_body,
        grid=(nb,),
        in_specs=[
            pl.BlockSpec((NBLK, 128), lambda i: (i, 0)),
            pl.BlockSpec((128, 128), lambda i: (0, 0)),
            pl.BlockSpec((1, 128), lambda i: (0, 0)),
            pl.BlockSpec((1, 128), lambda i: (0, 0)),
            pl.BlockSpec((1, 1), lambda i: (0, 0)),
        ],
        out_specs=[
            pl.BlockSpec((1, 1, NBLK), lambda i: (i, 0, 0)),
            pl.BlockSpec((1, 2), lambda i: (0, 0)),
        ],
        out_shape=[
            jax.ShapeDtypeStruct((nb, 1, NBLK), F32),
            jax.ShapeDtypeStruct((1, 2), F32),
        ],
        scratch_shapes=[pltpu.SMEM((1,), F32), pltpu.SMEM((1,), F32)],
    )(x, gate_w1, gate_b1.reshape(1, 128), gate_w2,
      gate_b2.reshape(1, 1))

    out = pl.pallas_call(
        _pool_body,
        grid=(nb,),
        in_specs=[
            pl.BlockSpec((NBLK, 128), lambda i: (i, 0)),
            pl.BlockSpec((1, 1, NBLK), lambda i: (i, 0, 0)),
            pl.BlockSpec((1, 1, NBLK), lambda i: (i, 0, 0)),
            pl.BlockSpec((1, 2), lambda i: (0, 0)),
            pl.BlockSpec((128, 128), lambda i: (0, 0)),
            pl.BlockSpec((1, 128), lambda i: (0, 0)),
            pl.BlockSpec((64, 128), lambda i: (0, 0)),
            pl.BlockSpec((1, 64), lambda i: (0, 0)),
            pl.BlockSpec((1, 64), lambda i: (0, 0)),
            pl.BlockSpec((1, 1), lambda i: (0, 0)),
        ],
        out_specs=pl.BlockSpec((1, 64), lambda i: (0, 0)),
        out_shape=jax.ShapeDtypeStruct((1, 64), F32),
        scratch_shapes=[pltpu.VMEM((64, 128), F32)],
    )(x, g3, batch.reshape(nb, 1, NBLK), gstats, mlp_w1,
      mlp_b1.reshape(1, 128), mlp_w2, mlp_b2.reshape(1, 64), mlp_w3,
      mlp_b3.reshape(1, 1))

    return out[0]


# speedup vs baseline: 68.2254x; 1.6442x over previous
import functools

import jax
import jax.numpy as jnp
from jax import lax
from jax.experimental import pallas as pl
from jax.experimental.pallas import tpu as pltpu
from jax.experimental.pallas import tpu_sc as plsc

F32 = jnp.float32
I32 = jnp.int32

NBLK = 1000
K = 80
SUPC = 5
NSUB = 16
NCORE = 2
NW = NSUB * NCORE


def _edge_stats_body(ew_ref, out_ref):
    ew = ew_ref[...]
    mn = jnp.min(ew)
    mx = jnp.max(ew)
    sm = jnp.sum(ew)
    lane = lax.broadcasted_iota(I32, (1, 128), 1)
    out_ref[...] = jnp.where(lane == 0, mn,
                             jnp.where(lane == 1, mx,
                                       jnp.where(lane == 2, sm, 0.0)))


def _in_proj_body(nf_ref, w_ref, b_ref, out_ref):
    out_ref[...] = lax.dot_general(
        nf_ref[...], w_ref[...], (((1,), (1,)), ((), ())),
        preferred_element_type=F32,
        precision=lax.Precision.HIGHEST) + b_ref[...]


def _proj_body(x_ref, lw_ref, aa_ref, xp_ref, asd_ref, mx_ref):
    i = pl.program_id(0)
    x = x_ref[...]
    xp_ref[...] = lax.dot_general(x, lw_ref[...], (((1,), (1,)), ((), ())),
                                  preferred_element_type=F32,
        precision=lax.Precision.HIGHEST)
    asd = lax.dot_general(x, aa_ref[...], (((1,), (0,)), ((), ())),
                          preferred_element_type=F32,
        precision=lax.Precision.HIGHEST)
    asd_ref[...] = asd
    bm = jnp.max(asd, axis=0, keepdims=True)

    @pl.when(i == 0)
    def _():
        mx_ref[...] = bm

    @pl.when(i > 0)
    def _():
        mx_ref[...] = jnp.maximum(mx_ref[...], bm)


def _combine_body(acc_ref, asd_ref, xp_ref, xres_ref, bias_ref, lg_ref,
                  lb_ref, consts_ref, out_ref, *, with_res):
    acc = acc_ref[...]
    a0 = acc[0]
    a1 = acc[1]
    out_raw = a0[:, :128] + a1[:, :128]
    den4 = a0[:, 128:132] + a1[:, 128:132]
    asd = asd_ref[...]
    a_s = asd[:, :4]
    a_d = asd[:, 4:8]
    cst = consts_ref[...]
    srow = cst[:, 0:4]
    cerow = cst[:, 4:8]
    fw = cst[0, 8]
    al = a_s + a_d + fw * cerow
    al = jnp.where(al >= 0, al, al * 0.2)
    exsl = jnp.exp(al - srow)
    den4 = den4 + exsl
    hh = lax.broadcasted_iota(I32, (4, 128), 0)
    ll = lax.broadcasted_iota(I32, (4, 128), 1) // 32
    rmat = jnp.where(hh == ll, 1.0, 0.0).astype(F32)
    den_w = lax.dot_general(den4, rmat, (((1,), (0,)), ((), ())),
                            preferred_element_type=F32,
        precision=lax.Precision.HIGHEST)
    exsl_w = lax.dot_general(exsl, rmat, (((1,), (0,)), ((), ())),
                             preferred_element_type=F32,
        precision=lax.Precision.HIGHEST)
    xp = xp_ref[...]
    out = (out_raw + exsl_w * xp) / (den_w + 1e-16) + bias_ref[...]
    h = jnp.where(out > 0, out, jnp.exp(jnp.minimum(out, 0.0)) - 1.0)
    m = jnp.mean(h, axis=1, keepdims=True)
    v = jnp.mean((h - m) ** 2, axis=1, keepdims=True)
    hn = (h - m) / jnp.sqrt(v + 1e-5) * lg_ref[...] + lb_ref[...]
    if with_res:
        hn = hn + xres_ref[...]
    out_ref[...] = hn


def _gate_body(x_ref, g1_ref, b1_ref, g2_ref, b2_ref, gout_ref, stats_ref,
               m_scr, z_scr):
    i = pl.program_id(0)
    nb = pl.num_programs(0)
    t = jnp.tanh(lax.dot_general(x_ref[...], g1_ref[...],
                                 (((1,), (1,)), ((), ())),
                                 preferred_element_type=F32,
        precision=lax.Precision.HIGHEST) + b1_ref[...])
    gb = lax.dot_general(g2_ref[...], t, (((1,), (1,)), ((), ())),
                         preferred_element_type=F32,
        precision=lax.Precision.HIGHEST) + b2_ref[0, 0]
    gout_ref[...] = gb[None]
    bm = jnp.max(gb)

    @pl.when(i == 0)
    def _():
        m_scr[0] = bm
        z_scr[0] = jnp.sum(jnp.exp(gb - bm))

    @pl.when(i > 0)
    def _():
        m_old = m_scr[0]
        m_new = jnp.maximum(m_old, bm)
        z_scr[0] = z_scr[0] * jnp.exp(m_old - m_new) + \
            jnp.sum(jnp.exp(gb - m_new))
        m_scr[0] = m_new

    @pl.when(i == nb - 1)
    def _():
        lane = lax.broadcasted_iota(I32, (1, 2), 1)
        stats_ref[...] = jnp.where(lane == 0, m_scr[0], z_scr[0])


def _pool_body(x_ref, g_ref, batch_ref, stats_ref, w1_ref, b1_ref, w2_ref,
               b2_ref, w3_ref, b3_ref, out_ref, pool_scr):
    i = pl.program_id(0)
    nb = pl.num_programs(0)
    blk = x_ref.shape[0]
    st = stats_ref[...]
    m = st[0, 0]
    z = st[0, 1]
    gb = g_ref[...][0]
    wgt = jnp.exp(gb - m) / z
    bb = batch_ref[...][0]
    rows = lax.broadcasted_iota(I32, (64, blk), 0)
    oh = jnp.where(rows == jnp.broadcast_to(bb, (64, blk)), 1.0, 0.0)
    ohw = oh.astype(F32) * wgt
    con = lax.dot_general(ohw, x_ref[...], (((1,), (0,)), ((), ())),
                          preferred_element_type=F32,
        precision=lax.Precision.HIGHEST)

    @pl.when(i == 0)
    def _():
        pool_scr[...] = con

    @pl.when(i > 0)
    def _():
        pool_scr[...] = pool_scr[...] + con

    @pl.when(i == nb - 1)
    def _():
        p = pool_scr[...]
        h1 = lax.dot_general(p, w1_ref[...], (((1,), (1,)), ((), ())),
                             preferred_element_type=F32,
        precision=lax.Precision.HIGHEST) + b1_ref[...]
        h1 = jnp.maximum(h1, 0.0)
        h2 = lax.dot_general(h1, w2_ref[...], (((1,), (1,)), ((), ())),
                             preferred_element_type=F32,
        precision=lax.Precision.HIGHEST) + b2_ref[...]
        h2 = jnp.maximum(h2, 0.0)
        o = lax.dot_general(w3_ref[...], h2, (((1,), (1,)), ((), ())),
                            preferred_element_type=F32,
        precision=lax.Precision.HIGHEST) + b3_ref[0, 0]
        out_ref[...] = o


def _sc_edge_body(edata_hbm, asd_hbm, xp_hbm, consts_hbm,
                  acc_hbm, consts_v, edata_v, dst_v, asr_a, asr_b, adr_a,
                  adr_b, rows_a, rows_b, rows144_v, acc_sp, sem_a, sem_b):
    e = edata_hbm.shape[1]
    npad = acc_hbm.shape[1]
    c = lax.axis_index("c")
    s = lax.axis_index("s")
    wid = s * NCORE + c
    epw = e // NW
    sup = SUPC * K
    nsup = epw // sup
    rpt = npad // NSUB

    pltpu.sync_copy(consts_hbm, consts_v)
    cv = consts_v[...]
    iota16 = jnp.arange(16, dtype=I32)

    z16 = jnp.zeros((16,), F32)
    cols = [(iota16 + 16 * j) for j in range(9)]

    def zrow(r, carry):
        rs = jnp.full((16,), r, I32)
        for j in range(9):
            plsc.store_scatter(rows144_v, [rs, cols[j]], z16)
        return carry

    lax.fori_loop(0, K, zrow, 0)

    r0 = s * rpt

    def zchunk(kk, carry):
        pltpu.sync_copy(rows144_v, acc_sp.at[pl.ds(r0 + kk * K, K)])
        return carry

    lax.fori_loop(0, rpt // K, zchunk, 0)
    plsc.subcore_barrier()

    ccols = [(iota16 + 32 * h + 16 * half)
             for h in range(4) for half in range(2)]
    ecols = [jnp.full((16,), 128 + h, I32) for h in range(4)]
    hrows = [jnp.full((16,), h, I32) for h in range(8)]
    bufs = [(asr_a, adr_a, rows_a, sem_a), (asr_b, adr_b, rows_b, sem_b)]

    def fire(j, buf):
        asr, adr, rows, sem = buf
        sidx = edata_v.at[0, pl.ds(j * K, K)]
        didx = edata_v.at[1, pl.ds(j * K, K)]
        return (pltpu.async_copy(asd_hbm.at[sidx], asr, sem),
                pltpu.async_copy(asd_hbm.at[didx], adr, sem),
                pltpu.async_copy(xp_hbm.at[sidx], rows, sem))

    def compute(j, buf):
        asr, adr, rows, _ = buf

        def group(g, gcarry):
            gi = g * 16 + iota16
            cpos = j * K + g * 16 + iota16
            dg = plsc.load_gather(edata_v, [hrows[1], cpos])
            wg = plsc.bitcast(plsc.load_gather(edata_v, [hrows[2], cpos]),
                              F32)
            plsc.store_scatter(dst_v, [gi], dg)
            exs = []
            for h in range(4):
                a_s = plsc.load_gather(asr, [gi, hrows[h]])
                a_d = plsc.load_gather(adr, [gi, hrows[4 + h]])
                al = a_s + a_d + wg * cv[4 + h]
                al = jnp.where(al >= 0, al, al * 0.2)
                ex = jnp.exp(al - cv[h])
                exs.append(ex)
                plsc.store_scatter(rows144_v, [gi, ecols[h]], ex)
            for ei in range(16):
                rs = jnp.full((16,), g * 16 + ei, I32)
                for h in range(4):
                    sc = exs[h][ei]
                    for half in range(2):
                        cc = ccols[h * 2 + half]
                        lg = plsc.load_gather(rows, [rs, cc])
                        plsc.store_scatter(rows144_v, [rs, cc], lg * sc)
            return gcarry

        lax.fori_loop(0, K // 16, group, 0)

    def super_body(sidx, carry):
        base = wid * epw + sidx * sup
        pltpu.sync_copy(edata_hbm.at[:, pl.ds(base, sup)], edata_v)
        descs = fire(0, bufs[0])
        for j in range(SUPC):
            cur = bufs[j % 2]
            if j + 1 < SUPC:
                nxt_descs = fire(j + 1, bufs[(j + 1) % 2])
            for d in descs:
                d.wait()
            compute(j, cur)
            pltpu.sync_copy(rows144_v, acc_sp.at[dst_v], add=True)
            if j + 1 < SUPC:
                descs = nxt_descs
        return carry

    lax.fori_loop(0, nsup, super_body, 0)
    plsc.subcore_barrier()
    pltpu.sync_copy(acc_sp.at[pl.ds(r0, rpt)],
                    acc_hbm.at[c, pl.ds(r0, rpt)])


def kernel(node_features, edge_index, edge_weight, batch, W_in, b_in, lin_w,
           att_src, att_dst, lin_edge_w, att_edge, gat_bias, ln_g, ln_b,
           gate_w1, gate_b1, gate_w2, gate_b2, mlp_w1, mlp_b1, mlp_w2,
           mlp_b2, mlp_w3, mlp_b3):
    n, _ = node_features.shape
    e = edge_weight.shape[0]
    num_layers, hid, _ = lin_w.shape
    nheads = att_src.shape[1]
    nb = n // NBLK
    src = edge_index[0]
    dst = edge_index[1]

    lw4 = lin_w.reshape(num_layers, nheads, hid // nheads, hid)
    fold_s = jnp.einsum('lhck,lhc->lkh', lw4, att_src)
    fold_d = jnp.einsum('lhck,lhc->lkh', lw4, att_dst)
    asad = jnp.concatenate([fold_s, fold_d], axis=2)
    ce = jnp.einsum('lhc,lhc->lh',
                    lin_edge_w.reshape(num_layers, nheads, hid // nheads),
                    att_edge)

    stats = pl.pallas_call(
        _edge_stats_body,
        grid=(1,),
        in_specs=[pl.BlockSpec((e // 128, 128), lambda i: (0, 0))],
        out_specs=pl.BlockSpec((1, 128), lambda i: (0, 0)),
        out_shape=jax.ShapeDtypeStruct((1, 128), F32),
    )(edge_weight.reshape(e // 128, 128))
    minw = stats[0, 0]
    maxw = stats[0, 1]
    fw = stats[0, 2] / e

    x = pl.pallas_call(
        _in_proj_body,
        grid=(nb,),
        in_specs=[
            pl.BlockSpec((NBLK, 128), lambda i: (i, 0)),
            pl.BlockSpec((128, 128), lambda i: (0, 0)),
            pl.BlockSpec((1, 128), lambda i: (0, 0)),
        ],
        out_specs=pl.BlockSpec((NBLK, 128), lambda i: (i, 0)),
        out_shape=jax.ShapeDtypeStruct((n, hid), F32),
    )(node_features, W_in, b_in.reshape(1, hid))

    edata = jnp.stack([src, dst,
                       lax.bitcast_convert_type(edge_weight, I32)])

    sc_call = pl.kernel(
        _sc_edge_body,
        out_type=jax.ShapeDtypeStruct((2, 10240, 144), F32),
        mesh=plsc.VectorSubcoreMesh(core_axis_name="c", subcore_axis_name="s"),
        compiler_params=pltpu.CompilerParams(use_tc_tiling_on_sc=False,
                                             needs_layout_passes=False),
        scratch_types=[
            pltpu.VMEM((16,), F32),
            pltpu.VMEM((3, SUPC * K), I32),
            pltpu.VMEM((K,), I32),
            pltpu.VMEM((K, 8), F32),
            pltpu.VMEM((K, 8), F32),
            pltpu.VMEM((K, 8), F32),
            pltpu.VMEM((K, 8), F32),
            pltpu.VMEM((K, 128), F32),
            pltpu.VMEM((K, 128), F32),
            pltpu.VMEM((K, 144), F32),
            pltpu.VMEM_SHARED((10240, 144), F32),
            pltpu.SemaphoreType.DMA,
            pltpu.SemaphoreType.DMA,
        ],
    )

    for i in range(num_layers):
        xp, asd, mx = pl.pallas_call(
            functools.partial(_proj_body),
            grid=(nb,),
            in_specs=[
                pl.BlockSpec((NBLK, 128), lambda i: (i, 0)),
                pl.BlockSpec((128, 128), lambda i: (0, 0)),
                pl.BlockSpec((128, 8), lambda i: (0, 0)),
            ],
            out_specs=[
                pl.BlockSpec((NBLK, 128), lambda i: (i, 0)),
                pl.BlockSpec((NBLK, 8), lambda i: (i, 0)),
                pl.BlockSpec((1, 8), lambda i: (0, 0)),
            ],
            out_shape=[
                jax.ShapeDtypeStruct((n, hid), F32),
                jax.ShapeDtypeStruct((n, 8), F32),
                jax.ShapeDtypeStruct((1, 8), F32),
            ],
        )(x, lin_w[i], asad[i])

        cei = ce[i]
        bound = mx[0, :4] + mx[0, 4:8] + jnp.maximum(
            jnp.maximum(cei * minw, cei * maxw), cei * fw)
        s = jnp.where(bound >= 0, bound, 0.2 * bound)
        consts = jnp.concatenate(
            [s, cei, jnp.stack([fw]), jnp.zeros((7,), F32)]).astype(F32)

        acc = sc_call(edata, asd, xp, consts)

        x = pl.pallas_call(
            functools.partial(_combine_body, with_res=(i > 0)),
            grid=(nb,),
            in_specs=[
                pl.BlockSpec((2, NBLK, 144), lambda i: (0, i, 0)),
                pl.BlockSpec((NBLK, 8), lambda i: (i, 0)),
                pl.BlockSpec((NBLK, 128), lambda i: (i, 0)),
                pl.BlockSpec((NBLK, 128), lambda i: (i, 0)),
                pl.BlockSpec((1, 128), lambda i: (0, 0)),
                pl.BlockSpec((1, 128), lambda i: (0, 0)),
                pl.BlockSpec((1, 128), lambda i: (0, 0)),
                pl.BlockSpec((1, 16), lambda i: (0, 0)),
            ],
            out_specs=pl.BlockSpec((NBLK, 128), lambda i: (i, 0)),
            out_shape=jax.ShapeDtypeStruct((n, hid), F32),
        )(acc, asd, xp, x, gat_bias[i].reshape(1, hid),
          ln_g[i].reshape(1, hid), ln_b[i].reshape(1, hid),
          consts.reshape(1, 16))

    g3, gstats = pl.pallas_call(
        _gate_body,
        grid=(nb,),
        in_specs=[
            pl.BlockSpec((NBLK, 128), lambda i: (i, 0)),
            pl.BlockSpec((128, 128), lambda i: (0, 0)),
            pl.BlockSpec((1, 128), lambda i: (0, 0)),
            pl.BlockSpec((1, 128), lambda i: (0, 0)),
            pl.BlockSpec((1, 1), lambda i: (0, 0)),
        ],
        out_specs=[
            pl.BlockSpec((1, 1, NBLK), lambda i: (i, 0, 0)),
            pl.BlockSpec((1, 2), lambda i: (0, 0)),
        ],
        out_shape=[
            jax.ShapeDtypeStruct((nb, 1, NBLK), F32),
            jax.ShapeDtypeStruct((1, 2), F32),
        ],
        scratch_shapes=[pltpu.SMEM((1,), F32), pltpu.SMEM((1,), F32)],
    )(x, gate_w1, gate_b1.reshape(1, 128), gate_w2,
      gate_b2.reshape(1, 1))

    out = pl.pallas_call(
        _pool_body,
        grid=(nb,),
        in_specs=[
            pl.BlockSpec((NBLK, 128), lambda i: (i, 0)),
            pl.BlockSpec((1, 1, NBLK), lambda i: (i, 0, 0)),
            pl.BlockSpec((1, 1, NBLK), lambda i: (i, 0, 0)),
            pl.BlockSpec((1, 2), lambda i: (0, 0)),
            pl.BlockSpec((128, 128), lambda i: (0, 0)),
            pl.BlockSpec((1, 128), lambda i: (0, 0)),
            pl.BlockSpec((64, 128), lambda i: (0, 0)),
            pl.BlockSpec((1, 64), lambda i: (0, 0)),
            pl.BlockSpec((1, 64), lambda i: (0, 0)),
            pl.BlockSpec((1, 1), lambda i: (0, 0)),
        ],
        out_specs=pl.BlockSpec((1, 64), lambda i: (0, 0)),
        out_shape=jax.ShapeDtypeStruct((1, 64), F32),
        scratch_shapes=[pltpu.VMEM((64, 128), F32)],
    )(x, g3, batch.reshape(nb, 1, NBLK), gstats, mlp_w1,
      mlp_b1.reshape(1, 128), mlp_w2, mlp_b2.reshape(1, 64), mlp_w3,
      mlp_b3.reshape(1, 1))

    return out[0]


# speedup vs baseline: 71.0281x; 1.0411x over previous
import functools

import jax
import jax.numpy as jnp
from jax import lax
from jax.experimental import pallas as pl
from jax.experimental.pallas import tpu as pltpu
from jax.experimental.pallas import tpu_sc as plsc

F32 = jnp.float32
I32 = jnp.int32

NBLK = 1000
K = 80
SUPC = 10
NSUB = 16
NCORE = 2
NW = NSUB * NCORE


def _edge_stats_body(ew_ref, out_ref):
    ew = ew_ref[...]
    mn = jnp.min(ew)
    mx = jnp.max(ew)
    sm = jnp.sum(ew)
    lane = lax.broadcasted_iota(I32, (1, 128), 1)
    out_ref[...] = jnp.where(lane == 0, mn,
                             jnp.where(lane == 1, mx,
                                       jnp.where(lane == 2, sm, 0.0)))


def _in_proj_body(nf_ref, w_ref, b_ref, out_ref):
    out_ref[...] = lax.dot_general(
        nf_ref[...], w_ref[...], (((1,), (1,)), ((), ())),
        preferred_element_type=F32,
        precision=lax.Precision.HIGHEST) + b_ref[...]


def _proj_body(x_ref, lw_ref, aa_ref, xp_ref, asd_ref, mx_ref):
    i = pl.program_id(0)
    x = x_ref[...]
    xp_ref[...] = lax.dot_general(x, lw_ref[...], (((1,), (1,)), ((), ())),
                                  preferred_element_type=F32,
        precision=lax.Precision.HIGHEST)
    asd = lax.dot_general(x, aa_ref[...], (((1,), (0,)), ((), ())),
                          preferred_element_type=F32,
        precision=lax.Precision.HIGHEST)
    asd_ref[...] = asd
    bm = jnp.max(asd, axis=0, keepdims=True)

    @pl.when(i == 0)
    def _():
        mx_ref[...] = bm

    @pl.when(i > 0)
    def _():
        mx_ref[...] = jnp.maximum(mx_ref[...], bm)


def _combine_body(acc_ref, asd_ref, xp_ref, xres_ref, bias_ref, lg_ref,
                  lb_ref, consts_ref, out_ref, *, with_res):
    acc = acc_ref[...]
    a0 = acc[0]
    a1 = acc[1]
    out_raw = a0[:, :128] + a1[:, :128]
    den4 = a0[:, 128:132] + a1[:, 128:132]
    asd = asd_ref[...]
    a_s = asd[:, :4]
    a_d = asd[:, 4:8]
    cst = consts_ref[...]
    srow = cst[:, 0:4]
    cerow = cst[:, 4:8]
    fw = cst[0, 8]
    al = a_s + a_d + fw * cerow
    al = jnp.where(al >= 0, al, al * 0.2)
    exsl = jnp.exp(al - srow)
    den4 = den4 + exsl
    hh = lax.broadcasted_iota(I32, (4, 128), 0)
    ll = lax.broadcasted_iota(I32, (4, 128), 1) // 32
    rmat = jnp.where(hh == ll, 1.0, 0.0).astype(F32)
    den_w = lax.dot_general(den4, rmat, (((1,), (0,)), ((), ())),
                            preferred_element_type=F32,
        precision=lax.Precision.HIGHEST)
    exsl_w = lax.dot_general(exsl, rmat, (((1,), (0,)), ((), ())),
                             preferred_element_type=F32,
        precision=lax.Precision.HIGHEST)
    xp = xp_ref[...]
    out = (out_raw + exsl_w * xp) / (den_w + 1e-16) + bias_ref[...]
    h = jnp.where(out > 0, out, jnp.exp(jnp.minimum(out, 0.0)) - 1.0)
    m = jnp.mean(h, axis=1, keepdims=True)
    v = jnp.mean((h - m) ** 2, axis=1, keepdims=True)
    hn = (h - m) / jnp.sqrt(v + 1e-5) * lg_ref[...] + lb_ref[...]
    if with_res:
        hn = hn + xres_ref[...]
    out_ref[...] = hn


def _gate_body(x_ref, g1_ref, b1_ref, g2_ref, b2_ref, gout_ref, stats_ref,
               m_scr, z_scr):
    i = pl.program_id(0)
    nb = pl.num_programs(0)
    t = jnp.tanh(lax.dot_general(x_ref[...], g1_ref[...],
                                 (((1,), (1,)), ((), ())),
                                 preferred_element_type=F32,
        precision=lax.Precision.HIGHEST) + b1_ref[...])
    gb = lax.dot_general(g2_ref[...], t, (((1,), (1,)), ((), ())),
                         preferred_element_type=F32,
        precision=lax.Precision.HIGHEST) + b2_ref[0, 0]
    gout_ref[...] = gb[None]
    bm = jnp.max(gb)

    @pl.when(i == 0)
    def _():
        m_scr[0] = bm
        z_scr[0] = jnp.sum(jnp.exp(gb - bm))

    @pl.when(i > 0)
    def _():
        m_old = m_scr[0]
        m_new = jnp.maximum(m_old, bm)
        z_scr[0] = z_scr[0] * jnp.exp(m_old - m_new) + \
            jnp.sum(jnp.exp(gb - m_new))
        m_scr[0] = m_new

    @pl.when(i == nb - 1)
    def _():
        lane = lax.broadcasted_iota(I32, (1, 2), 1)
        stats_ref[...] = jnp.where(lane == 0, m_scr[0], z_scr[0])


def _pool_body(x_ref, g_ref, batch_ref, stats_ref, w1_ref, b1_ref, w2_ref,
               b2_ref, w3_ref, b3_ref, out_ref, pool_scr):
    i = pl.program_id(0)
    nb = pl.num_programs(0)
    blk = x_ref.shape[0]
    st = stats_ref[...]
    m = st[0, 0]
    z = st[0, 1]
    gb = g_ref[...][0]
    wgt = jnp.exp(gb - m) / z
    bb = batch_ref[...][0]
    rows = lax.broadcasted_iota(I32, (64, blk), 0)
    oh = jnp.where(rows == jnp.broadcast_to(bb, (64, blk)), 1.0, 0.0)
    ohw = oh.astype(F32) * wgt
    con = lax.dot_general(ohw, x_ref[...], (((1,), (0,)), ((), ())),
                          preferred_element_type=F32,
        precision=lax.Precision.HIGHEST)

    @pl.when(i == 0)
    def _():
        pool_scr[...] = con

    @pl.when(i > 0)
    def _():
        pool_scr[...] = pool_scr[...] + con

    @pl.when(i == nb - 1)
    def _():
        p = pool_scr[...]
        h1 = lax.dot_general(p, w1_ref[...], (((1,), (1,)), ((), ())),
                             preferred_element_type=F32,
        precision=lax.Precision.HIGHEST) + b1_ref[...]
        h1 = jnp.maximum(h1, 0.0)
        h2 = lax.dot_general(h1, w2_ref[...], (((1,), (1,)), ((), ())),
                             preferred_element_type=F32,
        precision=lax.Precision.HIGHEST) + b2_ref[...]
        h2 = jnp.maximum(h2, 0.0)
        o = lax.dot_general(w3_ref[...], h2, (((1,), (1,)), ((), ())),
                            preferred_element_type=F32,
        precision=lax.Precision.HIGHEST) + b3_ref[0, 0]
        out_ref[...] = o


def _sc_edge_body(edata_hbm, asd_hbm, xp_hbm, consts_hbm,
                  acc_hbm, consts_v, edata_v, dst_v, asr_a, asr_b, adr_a,
                  adr_b, rows_a, rows_b, rows144_v, acc_sp, sem_a, sem_b):
    e = edata_hbm.shape[1]
    npad = acc_hbm.shape[1]
    c = lax.axis_index("c")
    s = lax.axis_index("s")
    wid = s * NCORE + c
    epw = e // NW
    sup = SUPC * K
    nsup = epw // sup
    rpt = npad // NSUB

    pltpu.sync_copy(consts_hbm, consts_v)
    cv = consts_v[...]
    iota16 = jnp.arange(16, dtype=I32)

    z16 = jnp.zeros((16,), F32)
    cols = [(iota16 + 16 * j) for j in range(9)]

    def zrow(r, carry):
        rs = jnp.full((16,), r, I32)
        for j in range(9):
            plsc.store_scatter(rows144_v, [rs, cols[j]], z16)
        return carry

    lax.fori_loop(0, K, zrow, 0)

    r0 = s * rpt

    def zchunk(kk, carry):
        pltpu.sync_copy(rows144_v, acc_sp.at[pl.ds(r0 + kk * K, K)])
        return carry

    lax.fori_loop(0, rpt // K, zchunk, 0)
    plsc.subcore_barrier()

    ccols = [(iota16 + 32 * h + 16 * half)
             for h in range(4) for half in range(2)]
    ecols = [jnp.full((16,), 128 + h, I32) for h in range(4)]
    hrows = [jnp.full((16,), h, I32) for h in range(8)]
    bufs = [(asr_a, adr_a, rows_a, sem_a), (asr_b, adr_b, rows_b, sem_b)]

    def fire(j, buf):
        asr, adr, rows, sem = buf
        sidx = edata_v.at[0, pl.ds(j * K, K)]
        didx = edata_v.at[1, pl.ds(j * K, K)]
        return (pltpu.async_copy(asd_hbm.at[sidx], asr, sem),
                pltpu.async_copy(asd_hbm.at[didx], adr, sem),
                pltpu.async_copy(xp_hbm.at[sidx], rows, sem))

    def compute(j, buf):
        asr, adr, rows, _ = buf

        def group(g, gcarry):
            gi = g * 16 + iota16
            cpos = j * K + g * 16 + iota16
            dg = plsc.load_gather(edata_v, [hrows[1], cpos])
            wg = plsc.bitcast(plsc.load_gather(edata_v, [hrows[2], cpos]),
                              F32)
            plsc.store_scatter(dst_v, [gi], dg)
            exs = []
            for h in range(4):
                a_s = plsc.load_gather(asr, [gi, hrows[h]])
                a_d = plsc.load_gather(adr, [gi, hrows[4 + h]])
                al = a_s + a_d + wg * cv[4 + h]
                al = jnp.where(al >= 0, al, al * 0.2)
                ex = jnp.exp(al - cv[h])
                exs.append(ex)
                plsc.store_scatter(rows144_v, [gi, ecols[h]], ex)
            for ei in range(16):
                rs = jnp.full((16,), g * 16 + ei, I32)
                for h in range(4):
                    sc = exs[h][ei]
                    for half in range(2):
                        cc = ccols[h * 2 + half]
                        lg = plsc.load_gather(rows, [rs, cc])
                        plsc.store_scatter(rows144_v, [rs, cc], lg * sc)
            return gcarry

        lax.fori_loop(0, K // 16, group, 0)

    def super_body(sidx, carry):
        base = wid * epw + sidx * sup
        pltpu.sync_copy(edata_hbm.at[:, pl.ds(base, sup)], edata_v)
        descs = fire(0, bufs[0])
        for j in range(SUPC):
            cur = bufs[j % 2]
            if j + 1 < SUPC:
                nxt_descs = fire(j + 1, bufs[(j + 1) % 2])
            for d in descs:
                d.wait()
            compute(j, cur)
            pltpu.sync_copy(rows144_v, acc_sp.at[dst_v], add=True)
            if j + 1 < SUPC:
                descs = nxt_descs
        return carry

    lax.fori_loop(0, nsup, super_body, 0)
    plsc.subcore_barrier()
    pltpu.sync_copy(acc_sp.at[pl.ds(r0, rpt)],
                    acc_hbm.at[c, pl.ds(r0, rpt)])


def kernel(node_features, edge_index, edge_weight, batch, W_in, b_in, lin_w,
           att_src, att_dst, lin_edge_w, att_edge, gat_bias, ln_g, ln_b,
           gate_w1, gate_b1, gate_w2, gate_b2, mlp_w1, mlp_b1, mlp_w2,
           mlp_b2, mlp_w3, mlp_b3):
    n, _ = node_features.shape
    e = edge_weight.shape[0]
    num_layers, hid, _ = lin_w.shape
    nheads = att_src.shape[1]
    nb = n // NBLK
    src = edge_index[0]
    dst = edge_index[1]

    lw4 = lin_w.reshape(num_layers, nheads, hid // nheads, hid)
    fold_s = jnp.einsum('lhck,lhc->lkh', lw4, att_src)
    fold_d = jnp.einsum('lhck,lhc->lkh', lw4, att_dst)
    asad = jnp.concatenate([fold_s, fold_d], axis=2)
    ce = jnp.einsum('lhc,lhc->lh',
                    lin_edge_w.reshape(num_layers, nheads, hid // nheads),
                    att_edge)

    stats = pl.pallas_call(
        _edge_stats_body,
        grid=(1,),
        in_specs=[pl.BlockSpec((e // 128, 128), lambda i: (0, 0))],
        out_specs=pl.BlockSpec((1, 128), lambda i: (0, 0)),
        out_shape=jax.ShapeDtypeStruct((1, 128), F32),
    )(edge_weight.reshape(e // 128, 128))
    minw = stats[0, 0]
    maxw = stats[0, 1]
    fw = stats[0, 2] / e

    x = pl.pallas_call(
        _in_proj_body,
        grid=(nb,),
        in_specs=[
            pl.BlockSpec((NBLK, 128), lambda i: (i, 0)),
            pl.BlockSpec((128, 128), lambda i: (0, 0)),
            pl.BlockSpec((1, 128), lambda i: (0, 0)),
        ],
        out_specs=pl.BlockSpec((NBLK, 128), lambda i: (i, 0)),
        out_shape=jax.ShapeDtypeStruct((n, hid), F32),
    )(node_features, W_in, b_in.reshape(1, hid))

    edata = jnp.stack([src, dst,
                       lax.bitcast_convert_type(edge_weight, I32)])

    sc_call = pl.kernel(
        _sc_edge_body,
        out_type=jax.ShapeDtypeStruct((2, 10240, 144), F32),
        mesh=plsc.VectorSubcoreMesh(core_axis_name="c", subcore_axis_name="s"),
        compiler_params=pltpu.CompilerParams(use_tc_tiling_on_sc=False,
                                             needs_layout_passes=False),
        scratch_types=[
            pltpu.VMEM((16,), F32),
            pltpu.VMEM((3, SUPC * K), I32),
            pltpu.VMEM((K,), I32),
            pltpu.VMEM((K, 8), F32),
            pltpu.VMEM((K, 8), F32),
            pltpu.VMEM((K, 8), F32),
            pltpu.VMEM((K, 8), F32),
            pltpu.VMEM((K, 128), F32),
            pltpu.VMEM((K, 128), F32),
            pltpu.VMEM((K, 144), F32),
            pltpu.VMEM_SHARED((10240, 144), F32),
            pltpu.SemaphoreType.DMA,
            pltpu.SemaphoreType.DMA,
        ],
    )

    for i in range(num_layers):
        xp, asd, mx = pl.pallas_call(
            functools.partial(_proj_body),
            grid=(nb,),
            in_specs=[
                pl.BlockSpec((NBLK, 128), lambda i: (i, 0)),
                pl.BlockSpec((128, 128), lambda i: (0, 0)),
                pl.BlockSpec((128, 8), lambda i: (0, 0)),
            ],
            out_specs=[
                pl.BlockSpec((NBLK, 128), lambda i: (i, 0)),
                pl.BlockSpec((NBLK, 8), lambda i: (i, 0)),
                pl.BlockSpec((1, 8), lambda i: (0, 0)),
            ],
            out_shape=[
                jax.ShapeDtypeStruct((n, hid), F32),
                jax.ShapeDtypeStruct((n, 8), F32),
                jax.ShapeDtypeStruct((1, 8), F32),
            ],
        )(x, lin_w[i], asad[i])

        cei = ce[i]
        bound = mx[0, :4] + mx[0, 4:8] + jnp.maximum(
            jnp.maximum(cei * minw, cei * maxw), cei * fw)
        s = jnp.where(bound >= 0, bound, 0.2 * bound)
        consts = jnp.concatenate(
            [s, cei, jnp.stack([fw]), jnp.zeros((7,), F32)]).astype(F32)

        acc = sc_call(edata, asd, xp, consts)

        x = pl.pallas_call(
            functools.partial(_combine_body, with_res=(i > 0)),
            grid=(nb,),
            in_specs=[
                pl.BlockSpec((2, NBLK, 144), lambda i: (0, i, 0)),
                pl.BlockSpec((NBLK, 8), lambda i: (i, 0)),
                pl.BlockSpec((NBLK, 128), lambda i: (i, 0)),
                pl.BlockSpec((NBLK, 128), lambda i: (i, 0)),
                pl.BlockSpec((1, 128), lambda i: (0, 0)),
                pl.BlockSpec((1, 128), lambda i: (0, 0)),
                pl.BlockSpec((1, 128), lambda i: (0, 0)),
                pl.BlockSpec((1, 16), lambda i: (0, 0)),
            ],
            out_specs=pl.BlockSpec((NBLK, 128), lambda i: (i, 0)),
            out_shape=jax.ShapeDtypeStruct((n, hid), F32),
        )(acc, asd, xp, x, gat_bias[i].reshape(1, hid),
          ln_g[i].reshape(1, hid), ln_b[i].reshape(1, hid),
          consts.reshape(1, 16))

    g3, gstats = pl.pallas_call(
        _gate_body,
        grid=(nb,),
        in_specs=[
            pl.BlockSpec((NBLK, 128), lambda i: (i, 0)),
            pl.BlockSpec((128, 128), lambda i: (0, 0)),
            pl.BlockSpec((1, 128), lambda i: (0, 0)),
            pl.BlockSpec((1, 128), lambda i: (0, 0)),
            pl.BlockSpec((1, 1), lambda i: (0, 0)),
        ],
        out_specs=[
            pl.BlockSpec((1, 1, NBLK), lambda i: (i, 0, 0)),
            pl.BlockSpec((1, 2), lambda i: (0, 0)),
        ],
        out_shape=[
            jax.ShapeDtypeStruct((nb, 1, NBLK), F32),
            jax.ShapeDtypeStruct((1, 2), F32),
        ],
        scratch_shapes=[pltpu.SMEM((1,), F32), pltpu.SMEM((1,), F32)],
    )(x, gate_w1, gate_b1.reshape(1, 128), gate_w2,
      gate_b2.reshape(1, 1))

    out = pl.pallas_call(
        _pool_body,
        grid=(nb,),
        in_specs=[
            pl.BlockSpec((NBLK, 128), lambda i: (i, 0)),
            pl.BlockSpec((1, 1, NBLK), lambda i: (i, 0, 0)),
            pl.BlockSpec((1, 1, NBLK), lambda i: (i, 0, 0)),
            pl.BlockSpec((1, 2), lambda i: (0, 0)),
            pl.BlockSpec((128, 128), lambda i: (0, 0)),
            pl.BlockSpec((1, 128), lambda i: (0, 0)),
            pl.BlockSpec((64, 128), lambda i: (0, 0)),
            pl.BlockSpec((1, 64), lambda i: (0, 0)),
            pl.BlockSpec((1, 64), lambda i: (0, 0)),
            pl.BlockSpec((1, 1), lambda i: (0, 0)),
        ],
        out_specs=pl.BlockSpec((1, 64), lambda i: (0, 0)),
        out_shape=jax.ShapeDtypeStruct((1, 64), F32),
        scratch_shapes=[pltpu.VMEM((64, 128), F32)],
    )(x, g3, batch.reshape(nb, 1, NBLK), gstats, mlp_w1,
      mlp_b1.reshape(1, 128), mlp_w2, mlp_b2.reshape(1, 64), mlp_w3,
      mlp_b3.reshape(1, 1))

    return out[0]


# speedup vs baseline: 71.3057x; 1.0039x over previous
import functools

import jax
import jax.numpy as jnp
from jax import lax
from jax.experimental import pallas as pl
from jax.experimental.pallas import tpu as pltpu
from jax.experimental.pallas import tpu_sc as plsc

F32 = jnp.float32
I32 = jnp.int32

NBLK = 1000
K = 80
SUPC = 10
NSUB = 16
NCORE = 2
NW = NSUB * NCORE


def _edge_stats_body(ew_ref, out_ref):
    ew = ew_ref[...]
    mn = jnp.min(ew)
    mx = jnp.max(ew)
    sm = jnp.sum(ew)
    lane = lax.broadcasted_iota(I32, (1, 128), 1)
    out_ref[...] = jnp.where(lane == 0, mn,
                             jnp.where(lane == 1, mx,
                                       jnp.where(lane == 2, sm, 0.0)))


def _in_proj_body(nf_ref, w_ref, b_ref, out_ref):
    out_ref[...] = lax.dot_general(
        nf_ref[...], w_ref[...], (((1,), (1,)), ((), ())),
        preferred_element_type=F32) + b_ref[...]


def _proj_body(x_ref, lw_ref, as_ref, ad_ref, xp_ref, asd_ref, mx_ref):
    i = pl.program_id(0)
    x = x_ref[...]
    xp = lax.dot_general(x, lw_ref[...], (((1,), (1,)), ((), ())),
                         preferred_element_type=F32)
    xp_ref[...] = xp
    kk = lax.broadcasted_iota(I32, (128, 4), 0) // 32
    hh = lax.broadcasted_iota(I32, (128, 4), 1)
    rt = jnp.where(kk == hh, 1.0, 0.0).astype(F32)
    a_s = lax.dot_general(xp * as_ref[...], rt, (((1,), (0,)), ((), ())),
                          preferred_element_type=F32,
                          precision=lax.Precision.HIGHEST)
    a_d = lax.dot_general(xp * ad_ref[...], rt, (((1,), (0,)), ((), ())),
                          preferred_element_type=F32,
                          precision=lax.Precision.HIGHEST)
    asd = jnp.concatenate([a_s, a_d], axis=1)
    asd_ref[...] = asd
    bm = jnp.max(asd, axis=0, keepdims=True)

    @pl.when(i == 0)
    def _():
        mx_ref[...] = bm

    @pl.when(i > 0)
    def _():
        mx_ref[...] = jnp.maximum(mx_ref[...], bm)


def _combine_body(acc_ref, asd_ref, xp_ref, xres_ref, bias_ref, lg_ref,
                  lb_ref, consts_ref, out_ref, *, with_res):
    acc = acc_ref[...]
    a0 = acc[0]
    a1 = acc[1]
    out_raw = a0[:, :128] + a1[:, :128]
    den4 = a0[:, 128:132] + a1[:, 128:132]
    asd = asd_ref[...]
    a_s = asd[:, :4]
    a_d = asd[:, 4:8]
    cst = consts_ref[...]
    srow = cst[:, 0:4]
    cerow = cst[:, 4:8]
    fw = cst[0, 8]
    al = a_s + a_d + fw * cerow
    al = jnp.where(al >= 0, al, al * 0.2)
    exsl = jnp.exp(al - srow)
    den4 = den4 + exsl
    hh = lax.broadcasted_iota(I32, (4, 128), 0)
    ll = lax.broadcasted_iota(I32, (4, 128), 1) // 32
    rmat = jnp.where(hh == ll, 1.0, 0.0).astype(F32)
    den_w = lax.dot_general(den4, rmat, (((1,), (0,)), ((), ())),
                            preferred_element_type=F32,
        precision=lax.Precision.HIGHEST)
    exsl_w = lax.dot_general(exsl, rmat, (((1,), (0,)), ((), ())),
                             preferred_element_type=F32,
        precision=lax.Precision.HIGHEST)
    xp = xp_ref[...]
    out = (out_raw + exsl_w * xp) / (den_w + 1e-16) + bias_ref[...]
    h = jnp.where(out > 0, out, jnp.exp(jnp.minimum(out, 0.0)) - 1.0)
    m = jnp.mean(h, axis=1, keepdims=True)
    v = jnp.mean((h - m) ** 2, axis=1, keepdims=True)
    hn = (h - m) / jnp.sqrt(v + 1e-5) * lg_ref[...] + lb_ref[...]
    if with_res:
        hn = hn + xres_ref[...]
    out_ref[...] = hn


def _gate_body(x_ref, g1_ref, b1_ref, g2_ref, b2_ref, gout_ref, stats_ref,
               m_scr, z_scr):
    i = pl.program_id(0)
    nb = pl.num_programs(0)
    t = jnp.tanh(lax.dot_general(x_ref[...], g1_ref[...],
                                 (((1,), (1,)), ((), ())),
                                 preferred_element_type=F32) + b1_ref[...])
    tb = t.astype(jnp.bfloat16).astype(F32)
    gb = jnp.sum(tb * g2_ref[...], axis=1, keepdims=True) + b2_ref[0, 0]
    gout_ref[...] = gb
    bm = jnp.max(gb)

    @pl.when(i == 0)
    def _():
        m_scr[0] = bm
        z_scr[0] = jnp.sum(jnp.exp(gb - bm))

    @pl.when(i > 0)
    def _():
        m_old = m_scr[0]
        m_new = jnp.maximum(m_old, bm)
        z_scr[0] = z_scr[0] * jnp.exp(m_old - m_new) + \
            jnp.sum(jnp.exp(gb - m_new))
        m_scr[0] = m_new

    @pl.when(i == nb - 1)
    def _():
        lane = lax.broadcasted_iota(I32, (1, 2), 1)
        stats_ref[...] = jnp.where(lane == 0, m_scr[0], z_scr[0])


def _pool_body(x_ref, g_ref, batch_ref, stats_ref, w1_ref, b1_ref, w2_ref,
               b2_ref, w3_ref, b3_ref, out_ref, pool_scr):
    i = pl.program_id(0)
    nb = pl.num_programs(0)
    blk = x_ref.shape[0]
    st = stats_ref[...]
    m = st[0, 0]
    z = st[0, 1]
    gb = g_ref[...]
    wgt = jnp.exp(gb - m) / z
    xw = x_ref[...] * wgt
    bb = batch_ref[...][0]
    rows = lax.broadcasted_iota(I32, (64, blk), 0)
    oh = jnp.where(rows == jnp.broadcast_to(bb, (64, blk)), 1.0, 0.0)
    con = lax.dot_general(oh.astype(F32), xw, (((1,), (0,)), ((), ())),
                          preferred_element_type=F32,
        precision=lax.Precision.HIGHEST)

    @pl.when(i == 0)
    def _():
        pool_scr[...] = con

    @pl.when(i > 0)
    def _():
        pool_scr[...] = pool_scr[...] + con

    @pl.when(i == nb - 1)
    def _():
        p = pool_scr[...]
        h1 = lax.dot_general(p, w1_ref[...], (((1,), (1,)), ((), ())),
                             preferred_element_type=F32) + b1_ref[...]
        h1 = jnp.maximum(h1, 0.0)
        h2 = lax.dot_general(h1, w2_ref[...], (((1,), (1,)), ((), ())),
                             preferred_element_type=F32) + b2_ref[...]
        h2 = jnp.maximum(h2, 0.0)
        o = lax.dot_general(w3_ref[...], h2, (((1,), (1,)), ((), ())),
                            preferred_element_type=F32) + b3_ref[0, 0]
        out_ref[...] = o


def _sc_edge_body(edata_hbm, asd_hbm, xp_hbm, consts_hbm,
                  acc_hbm, consts_v, edata_v, dst_v, asr_a, asr_b, adr_a,
                  adr_b, rows_a, rows_b, rows144_v, acc_sp, sem_a, sem_b):
    e = edata_hbm.shape[1]
    npad = acc_hbm.shape[1]
    c = lax.axis_index("c")
    s = lax.axis_index("s")
    wid = s * NCORE + c
    epw = e // NW
    sup = SUPC * K
    nsup = epw // sup
    rpt = npad // NSUB

    pltpu.sync_copy(consts_hbm, consts_v)
    cv = consts_v[...]
    iota16 = jnp.arange(16, dtype=I32)

    z16 = jnp.zeros((16,), F32)
    cols = [(iota16 + 16 * j) for j in range(9)]

    def zrow(r, carry):
        rs = jnp.full((16,), r, I32)
        for j in range(9):
            plsc.store_scatter(rows144_v, [rs, cols[j]], z16)
        return carry

    lax.fori_loop(0, K, zrow, 0)

    r0 = s * rpt

    def zchunk(kk, carry):
        pltpu.sync_copy(rows144_v, acc_sp.at[pl.ds(r0 + kk * K, K)])
        return carry

    lax.fori_loop(0, rpt // K, zchunk, 0)
    plsc.subcore_barrier()

    ccols = [(iota16 + 32 * h + 16 * half)
             for h in range(4) for half in range(2)]
    ecols = [jnp.full((16,), 128 + h, I32) for h in range(4)]
    hrows = [jnp.full((16,), h, I32) for h in range(8)]
    bufs = [(asr_a, adr_a, rows_a, sem_a), (asr_b, adr_b, rows_b, sem_b)]

    def fire(j, buf):
        asr, adr, rows, sem = buf
        sidx = edata_v.at[0, pl.ds(j * K, K)]
        didx = edata_v.at[1, pl.ds(j * K, K)]
        return (pltpu.async_copy(asd_hbm.at[sidx], asr, sem),
                pltpu.async_copy(asd_hbm.at[didx], adr, sem),
                pltpu.async_copy(xp_hbm.at[sidx], rows, sem))

    def compute(j, buf):
        asr, adr, rows, _ = buf

        def group(g, gcarry):
            gi = g * 16 + iota16
            cpos = j * K + g * 16 + iota16
            dg = plsc.load_gather(edata_v, [hrows[1], cpos])
            wg = plsc.bitcast(plsc.load_gather(edata_v, [hrows[2], cpos]),
                              F32)
            plsc.store_scatter(dst_v, [gi], dg)
            exs = []
            for h in range(4):
                a_s = plsc.load_gather(asr, [gi, hrows[h]])
                a_d = plsc.load_gather(adr, [gi, hrows[4 + h]])
                al = a_s + a_d + wg * cv[4 + h]
                al = jnp.where(al >= 0, al, al * 0.2)
                ex = jnp.exp(al - cv[h])
                exs.append(ex)
                plsc.store_scatter(rows144_v, [gi, ecols[h]], ex)
            for ei in range(16):
                rs = jnp.full((16,), g * 16 + ei, I32)
                for h in range(4):
                    sc = exs[h][ei]
                    for half in range(2):
                        cc = ccols[h * 2 + half]
                        lg = plsc.load_gather(rows, [rs, cc])
                        plsc.store_scatter(rows144_v, [rs, cc], lg * sc)
            return gcarry

        lax.fori_loop(0, K // 16, group, 0)

    def super_body(sidx, carry):
        base = wid * epw + sidx * sup
        pltpu.sync_copy(edata_hbm.at[:, pl.ds(base, sup)], edata_v)
        descs = fire(0, bufs[0])
        for j in range(SUPC):
            cur = bufs[j % 2]
            if j + 1 < SUPC:
                nxt_descs = fire(j + 1, bufs[(j + 1) % 2])
            for d in descs:
                d.wait()
            compute(j, cur)
            pltpu.sync_copy(rows144_v, acc_sp.at[dst_v], add=True)
            if j + 1 < SUPC:
                descs = nxt_descs
        return carry

    lax.fori_loop(0, nsup, super_body, 0)
    plsc.subcore_barrier()
    pltpu.sync_copy(acc_sp.at[pl.ds(r0, rpt)],
                    acc_hbm.at[c, pl.ds(r0, rpt)])


def kernel(node_features, edge_index, edge_weight, batch, W_in, b_in, lin_w,
           att_src, att_dst, lin_edge_w, att_edge, gat_bias, ln_g, ln_b,
           gate_w1, gate_b1, gate_w2, gate_b2, mlp_w1, mlp_b1, mlp_w2,
           mlp_b2, mlp_w3, mlp_b3):
    n, _ = node_features.shape
    e = edge_weight.shape[0]
    num_layers, hid, _ = lin_w.shape
    nheads = att_src.shape[1]
    nb = n // NBLK
    src = edge_index[0]
    dst = edge_index[1]

    lewb = lin_edge_w.astype(jnp.bfloat16).astype(F32)
    ce = jnp.einsum('lhc,lhc->lh',
                    lewb.reshape(num_layers, nheads, hid // nheads),
                    att_edge,
                    precision=lax.Precision.HIGHEST)
    wq = edge_weight.astype(jnp.bfloat16).astype(F32)

    stats = pl.pallas_call(
        _edge_stats_body,
        grid=(1,),
        in_specs=[pl.BlockSpec((e // 128, 128), lambda i: (0, 0))],
        out_specs=pl.BlockSpec((1, 128), lambda i: (0, 0)),
        out_shape=jax.ShapeDtypeStruct((1, 128), F32),
    )(edge_weight.reshape(e // 128, 128))
    minw = stats[0, 0].astype(jnp.bfloat16).astype(F32)
    maxw = stats[0, 1].astype(jnp.bfloat16).astype(F32)
    fw = (stats[0, 2] / e).astype(jnp.bfloat16).astype(F32)

    x = pl.pallas_call(
        _in_proj_body,
        grid=(nb,),
        in_specs=[
            pl.BlockSpec((NBLK, 128), lambda i: (i, 0)),
            pl.BlockSpec((128, 128), lambda i: (0, 0)),
            pl.BlockSpec((1, 128), lambda i: (0, 0)),
        ],
        out_specs=pl.BlockSpec((NBLK, 128), lambda i: (i, 0)),
        out_shape=jax.ShapeDtypeStruct((n, hid), F32),
    )(node_features, W_in, b_in.reshape(1, hid))

    edata = jnp.stack([src, dst,
                       lax.bitcast_convert_type(wq, I32)])

    sc_call = pl.kernel(
        _sc_edge_body,
        out_type=jax.ShapeDtypeStruct((2, 10240, 144), F32),
        mesh=plsc.VectorSubcoreMesh(core_axis_name="c", subcore_axis_name="s"),
        compiler_params=pltpu.CompilerParams(use_tc_tiling_on_sc=False,
                                             needs_layout_passes=False),
        scratch_types=[
            pltpu.VMEM((16,), F32),
            pltpu.VMEM((3, SUPC * K), I32),
            pltpu.VMEM((K,), I32),
            pltpu.VMEM((K, 8), F32),
            pltpu.VMEM((K, 8), F32),
            pltpu.VMEM((K, 8), F32),
            pltpu.VMEM((K, 8), F32),
            pltpu.VMEM((K, 128), F32),
            pltpu.VMEM((K, 128), F32),
            pltpu.VMEM((K, 144), F32),
            pltpu.VMEM_SHARED((10240, 144), F32),
            pltpu.SemaphoreType.DMA,
            pltpu.SemaphoreType.DMA,
        ],
    )

    for i in range(num_layers):
        xp, asd, mx = pl.pallas_call(
            functools.partial(_proj_body),
            grid=(nb,),
            in_specs=[
                pl.BlockSpec((NBLK, 128), lambda i: (i, 0)),
                pl.BlockSpec((128, 128), lambda i: (0, 0)),
                pl.BlockSpec((1, 128), lambda i: (0, 0)),
                pl.BlockSpec((1, 128), lambda i: (0, 0)),
            ],
            out_specs=[
                pl.BlockSpec((NBLK, 128), lambda i: (i, 0)),
                pl.BlockSpec((NBLK, 8), lambda i: (i, 0)),
                pl.BlockSpec((1, 8), lambda i: (0, 0)),
            ],
            out_shape=[
                jax.ShapeDtypeStruct((n, hid), F32),
                jax.ShapeDtypeStruct((n, 8), F32),
                jax.ShapeDtypeStruct((1, 8), F32),
            ],
        )(x, lin_w[i], att_src[i].reshape(1, hid), att_dst[i].reshape(1, hid))

        cei = ce[i]
        bound = mx[0, :4] + mx[0, 4:8] + jnp.maximum(
            jnp.maximum(cei * minw, cei * maxw), cei * fw)
        s = jnp.where(bound >= 0, bound, 0.2 * bound)
        consts = jnp.concatenate(
            [s, cei, jnp.stack([fw]), jnp.zeros((7,), F32)]).astype(F32)

        acc = sc_call(edata, asd, xp, consts)

        x = pl.pallas_call(
            functools.partial(_combine_body, with_res=(i > 0)),
            grid=(nb,),
            in_specs=[
                pl.BlockSpec((2, NBLK, 144), lambda i: (0, i, 0)),
                pl.BlockSpec((NBLK, 8), lambda i: (i, 0)),
                pl.BlockSpec((NBLK, 128), lambda i: (i, 0)),
                pl.BlockSpec((NBLK, 128), lambda i: (i, 0)),
                pl.BlockSpec((1, 128), lambda i: (0, 0)),
                pl.BlockSpec((1, 128), lambda i: (0, 0)),
                pl.BlockSpec((1, 128), lambda i: (0, 0)),
                pl.BlockSpec((1, 16), lambda i: (0, 0)),
            ],
            out_specs=pl.BlockSpec((NBLK, 128), lambda i: (i, 0)),
            out_shape=jax.ShapeDtypeStruct((n, hid), F32),
        )(acc, asd, xp, x, gat_bias[i].reshape(1, hid),
          ln_g[i].reshape(1, hid), ln_b[i].reshape(1, hid),
          consts.reshape(1, 16))

    g3, gstats = pl.pallas_call(
        _gate_body,
        grid=(nb,),
        in_specs=[
            pl.BlockSpec((NBLK, 128), lambda i: (i, 0)),
            pl.BlockSpec((128, 128), lambda i: (0, 0)),
            pl.BlockSpec((1, 128), lambda i: (0, 0)),
            pl.BlockSpec((1, 128), lambda i: (0, 0)),
            pl.BlockSpec((1, 1), lambda i: (0, 0)),
        ],
        out_specs=[
            pl.BlockSpec((NBLK, 1), lambda i: (i, 0)),
            pl.BlockSpec((1, 2), lambda i: (0, 0)),
        ],
        out_shape=[
            jax.ShapeDtypeStruct((n, 1), F32),
            jax.ShapeDtypeStruct((1, 2), F32),
        ],
        scratch_shapes=[pltpu.SMEM((1,), F32), pltpu.SMEM((1,), F32)],
    )(x, gate_w1, gate_b1.reshape(1, 128),
      gate_w2.astype(jnp.bfloat16).astype(F32),
      gate_b2.reshape(1, 1))

    out = pl.pallas_call(
        _pool_body,
        grid=(nb,),
        in_specs=[
            pl.BlockSpec((NBLK, 128), lambda i: (i, 0)),
            pl.BlockSpec((NBLK, 1), lambda i: (i, 0)),
            pl.BlockSpec((1, 1, NBLK), lambda i: (i, 0, 0)),
            pl.BlockSpec((1, 2), lambda i: (0, 0)),
            pl.BlockSpec((128, 128), lambda i: (0, 0)),
            pl.BlockSpec((1, 128), lambda i: (0, 0)),
            pl.BlockSpec((64, 128), lambda i: (0, 0)),
            pl.BlockSpec((1, 64), lambda i: (0, 0)),
            pl.BlockSpec((1, 64), lambda i: (0, 0)),
            pl.BlockSpec((1, 1), lambda i: (0, 0)),
        ],
        out_specs=pl.BlockSpec((1, 64), lambda i: (0, 0)),
        out_shape=jax.ShapeDtypeStruct((1, 64), F32),
        scratch_shapes=[pltpu.VMEM((64, 128), F32)],
    )(x, g3, batch.reshape(nb, 1, NBLK), gstats, mlp_w1,
      mlp_b1.reshape(1, 128), mlp_w2, mlp_b2.reshape(1, 64), mlp_w3,
      mlp_b3.reshape(1, 1))

    return out[0]


# speedup vs baseline: 71.8640x; 1.0078x over previous
import functools

import jax
import jax.numpy as jnp
from jax import lax
from jax.experimental import pallas as pl
from jax.experimental.pallas import tpu as pltpu
from jax.experimental.pallas import tpu_sc as plsc

F32 = jnp.float32
I32 = jnp.int32

NBLK = 1000
K = 80
SUPC = 10
NSUB = 16
NCORE = 2
NW = NSUB * NCORE


def _edge_stats_body(ew_ref, out_ref):
    ew = ew_ref[...]
    mn = jnp.min(ew)
    mx = jnp.max(ew)
    sm = jnp.sum(ew)
    lane = lax.broadcasted_iota(I32, (1, 128), 1)
    out_ref[...] = jnp.where(lane == 0, mn,
                             jnp.where(lane == 1, mx,
                                       jnp.where(lane == 2, sm, 0.0)))


def _in_proj_body(nf_ref, w_ref, b_ref, out_ref):
    out_ref[...] = lax.dot_general(
        nf_ref[...], w_ref[...], (((1,), (1,)), ((), ())),
        preferred_element_type=F32) + b_ref[...]


def _proj_body(x_ref, lw_ref, as_ref, ad_ref, xp_ref, asd_ref, mx_ref):
    i = pl.program_id(0)
    x = x_ref[...]
    xp = lax.dot_general(x, lw_ref[...], (((1,), (1,)), ((), ())),
                         preferred_element_type=F32)
    xp_ref[...] = xp
    kk = lax.broadcasted_iota(I32, (128, 4), 0) // 32
    hh = lax.broadcasted_iota(I32, (128, 4), 1)
    rt = jnp.where(kk == hh, 1.0, 0.0).astype(F32)
    a_s = lax.dot_general(xp * as_ref[...], rt, (((1,), (0,)), ((), ())),
                          preferred_element_type=F32,
                          precision=lax.Precision.HIGHEST)
    a_d = lax.dot_general(xp * ad_ref[...], rt, (((1,), (0,)), ((), ())),
                          preferred_element_type=F32,
                          precision=lax.Precision.HIGHEST)
    asd = jnp.concatenate([a_s, a_d], axis=1)
    asd_ref[...] = asd
    bm = jnp.max(asd, axis=0, keepdims=True)

    @pl.when(i == 0)
    def _():
        mx_ref[...] = bm

    @pl.when(i > 0)
    def _():
        mx_ref[...] = jnp.maximum(mx_ref[...], bm)


def _combine_body(acc_ref, asd_ref, xp_ref, xres_ref, bias_ref, lg_ref,
                  lb_ref, consts_ref, out_ref, *, with_res):
    acc = acc_ref[...]
    a0 = acc[0]
    a1 = acc[1]
    out_raw = a0[:, :128] + a1[:, :128]
    den4 = a0[:, 128:132] + a1[:, 128:132]
    asd = asd_ref[...]
    a_s = asd[:, :4]
    a_d = asd[:, 4:8]
    cst = consts_ref[...]
    srow = cst[:, 0:4]
    cerow = cst[:, 4:8]
    fw = cst[0, 8]
    al = a_s + a_d + fw * cerow
    al = jnp.where(al >= 0, al, al * 0.2)
    exsl = jnp.exp(al - srow)
    den4 = den4 + exsl
    hh = lax.broadcasted_iota(I32, (4, 128), 0)
    ll = lax.broadcasted_iota(I32, (4, 128), 1) // 32
    rmat = jnp.where(hh == ll, 1.0, 0.0).astype(F32)
    den_w = lax.dot_general(den4, rmat, (((1,), (0,)), ((), ())),
                            preferred_element_type=F32,
        precision=lax.Precision.HIGHEST)
    exsl_w = lax.dot_general(exsl, rmat, (((1,), (0,)), ((), ())),
                             preferred_element_type=F32,
        precision=lax.Precision.HIGHEST)
    xp = xp_ref[...]
    out = (out_raw + exsl_w * xp) / (den_w + 1e-16) + bias_ref[...]
    h = jnp.where(out > 0, out, jnp.exp(jnp.minimum(out, 0.0)) - 1.0)
    m = jnp.mean(h, axis=1, keepdims=True)
    v = jnp.mean((h - m) ** 2, axis=1, keepdims=True)
    hn = (h - m) / jnp.sqrt(v + 1e-5) * lg_ref[...] + lb_ref[...]
    if with_res:
        hn = hn + xres_ref[...]
    out_ref[...] = hn


def _gate_body(x_ref, g1_ref, b1_ref, g2_ref, b2_ref, gout_ref, stats_ref,
               m_scr, z_scr):
    i = pl.program_id(0)
    nb = pl.num_programs(0)
    t = jnp.tanh(lax.dot_general(x_ref[...], g1_ref[...],
                                 (((1,), (1,)), ((), ())),
                                 preferred_element_type=F32) + b1_ref[...])
    tb = t.astype(jnp.bfloat16).astype(F32)
    gb = jnp.sum(tb * g2_ref[...], axis=1, keepdims=True) + b2_ref[0, 0]
    gout_ref[...] = gb
    bm = jnp.max(gb)

    @pl.when(i == 0)
    def _():
        m_scr[0] = bm
        z_scr[0] = jnp.sum(jnp.exp(gb - bm))

    @pl.when(i > 0)
    def _():
        m_old = m_scr[0]
        m_new = jnp.maximum(m_old, bm)
        z_scr[0] = z_scr[0] * jnp.exp(m_old - m_new) + \
            jnp.sum(jnp.exp(gb - m_new))
        m_scr[0] = m_new

    @pl.when(i == nb - 1)
    def _():
        lane = lax.broadcasted_iota(I32, (1, 2), 1)
        stats_ref[...] = jnp.where(lane == 0, m_scr[0], z_scr[0])


def _pool_body(x_ref, g_ref, batch_ref, stats_ref, w1_ref, b1_ref, w2_ref,
               b2_ref, w3_ref, b3_ref, out_ref, pool_scr):
    i = pl.program_id(0)
    nb = pl.num_programs(0)
    blk = x_ref.shape[0]
    st = stats_ref[...]
    m = st[0, 0]
    z = st[0, 1]
    gb = g_ref[...]
    wgt = jnp.exp(gb - m) / z
    xw = x_ref[...] * wgt
    bb = batch_ref[...][0]
    rows = lax.broadcasted_iota(I32, (64, blk), 0)
    oh = jnp.where(rows == jnp.broadcast_to(bb, (64, blk)), 1.0, 0.0)
    con = lax.dot_general(oh.astype(F32), xw, (((1,), (0,)), ((), ())),
                          preferred_element_type=F32,
        precision=lax.Precision.HIGHEST)

    @pl.when(i == 0)
    def _():
        pool_scr[...] = con

    @pl.when(i > 0)
    def _():
        pool_scr[...] = pool_scr[...] + con

    @pl.when(i == nb - 1)
    def _():
        p = pool_scr[...]
        h1 = lax.dot_general(p, w1_ref[...], (((1,), (1,)), ((), ())),
                             preferred_element_type=F32) + b1_ref[...]
        h1 = jnp.maximum(h1, 0.0)
        h2 = lax.dot_general(h1, w2_ref[...], (((1,), (1,)), ((), ())),
                             preferred_element_type=F32) + b2_ref[...]
        h2 = jnp.maximum(h2, 0.0)
        o = lax.dot_general(w3_ref[...], h2, (((1,), (1,)), ((), ())),
                            preferred_element_type=F32) + b3_ref[0, 0]
        out_ref[...] = o


def _sc_edge_body(edata_hbm, asd_hbm, xp_hbm, consts_hbm,
                  acc_hbm, consts_v, edata_v, dst_v, asr_a, asr_b, adr_a,
                  adr_b, rows_a, rows_b, rows144_v, acc_sp, sem_a, sem_b):
    e = edata_hbm.shape[1]
    npad = acc_hbm.shape[1]
    c = lax.axis_index("c")
    s = lax.axis_index("s")
    wid = s * NCORE + c
    epw = e // NW
    sup = SUPC * K
    nsup = epw // sup
    rpt = npad // NSUB

    pltpu.sync_copy(consts_hbm, consts_v)
    cv = consts_v[...]
    iota16 = jnp.arange(16, dtype=I32)

    z16 = jnp.zeros((16,), F32)
    cols = [(iota16 + 16 * j) for j in range(9)]

    def zrow(r, carry):
        rs = jnp.full((16,), r, I32)
        for j in range(9):
            plsc.store_scatter(rows144_v, [rs, cols[j]], z16)
        return carry

    lax.fori_loop(0, K, zrow, 0)

    r0 = s * rpt

    def zchunk(kk, carry):
        pltpu.sync_copy(rows144_v, acc_sp.at[pl.ds(r0 + kk * K, K)])
        return carry

    lax.fori_loop(0, rpt // K, zchunk, 0)
    plsc.subcore_barrier()

    ccols = [(iota16 + 32 * h + 16 * half)
             for h in range(4) for half in range(2)]
    ecols = [jnp.full((16,), 128 + h, I32) for h in range(4)]
    hrows = [jnp.full((16,), h, I32) for h in range(8)]
    bufs = [(asr_a, adr_a, rows_a, sem_a), (asr_b, adr_b, rows_b, sem_b)]

    def fire(j, buf):
        asr, adr, rows, sem = buf
        sidx = edata_v.at[0, pl.ds(j * K, K)]
        didx = edata_v.at[1, pl.ds(j * K, K)]
        return (pltpu.async_copy(asd_hbm.at[sidx], asr, sem),
                pltpu.async_copy(asd_hbm.at[didx], adr, sem),
                pltpu.async_copy(xp_hbm.at[sidx], rows, sem))

    def compute(j, buf):
        asr, adr, rows, _ = buf

        def group(g, gcarry):
            gi = g * 16 + iota16
            cpos = j * K + g * 16 + iota16
            dg = plsc.load_gather(edata_v, [hrows[1], cpos])
            wg = plsc.bitcast(plsc.load_gather(edata_v, [hrows[2], cpos]),
                              F32)
            plsc.store_scatter(dst_v, [gi], dg)
            exs = []
            for h in range(4):
                a_s = plsc.load_gather(asr, [gi, hrows[h]])
                a_d = plsc.load_gather(adr, [gi, hrows[4 + h]])
                al = a_s + a_d + wg * cv[4 + h]
                al = jnp.where(al >= 0, al, al * 0.2)
                ex = jnp.exp(al - cv[h])
                exs.append(ex)
                plsc.store_scatter(rows144_v, [gi, ecols[h]], ex)
            for ei in range(16):
                r = g * 16 + ei
                for h in range(4):
                    sc = exs[h][ei]
                    for half in range(2):
                        cof = 32 * h + 16 * half
                        lg = rows[r, pl.ds(cof, 16)]
                        rows144_v[r, pl.ds(cof, 16)] = lg * sc
            return gcarry

        lax.fori_loop(0, K // 16, group, 0)

    def super_body(sidx, carry):
        base = wid * epw + sidx * sup
        pltpu.sync_copy(edata_hbm.at[:, pl.ds(base, sup)], edata_v)
        descs = fire(0, bufs[0])
        for j in range(SUPC):
            cur = bufs[j % 2]
            if j + 1 < SUPC:
                nxt_descs = fire(j + 1, bufs[(j + 1) % 2])
            for d in descs:
                d.wait()
            compute(j, cur)
            pltpu.sync_copy(rows144_v, acc_sp.at[dst_v], add=True)
            if j + 1 < SUPC:
                descs = nxt_descs
        return carry

    lax.fori_loop(0, nsup, super_body, 0)
    plsc.subcore_barrier()
    pltpu.sync_copy(acc_sp.at[pl.ds(r0, rpt)],
                    acc_hbm.at[c, pl.ds(r0, rpt)])


def kernel(node_features, edge_index, edge_weight, batch, W_in, b_in, lin_w,
           att_src, att_dst, lin_edge_w, att_edge, gat_bias, ln_g, ln_b,
           gate_w1, gate_b1, gate_w2, gate_b2, mlp_w1, mlp_b1, mlp_w2,
           mlp_b2, mlp_w3, mlp_b3):
    n, _ = node_features.shape
    e = edge_weight.shape[0]
    num_layers, hid, _ = lin_w.shape
    nheads = att_src.shape[1]
    nb = n // NBLK
    src = edge_index[0]
    dst = edge_index[1]

    lewb = lin_edge_w.astype(jnp.bfloat16).astype(F32)
    ce = jnp.einsum('lhc,lhc->lh',
                    lewb.reshape(num_layers, nheads, hid // nheads),
                    att_edge,
                    precision=lax.Precision.HIGHEST)
    wq = edge_weight.astype(jnp.bfloat16).astype(F32)

    stats = pl.pallas_call(
        _edge_stats_body,
        grid=(1,),
        in_specs=[pl.BlockSpec((e // 128, 128), lambda i: (0, 0))],
        out_specs=pl.BlockSpec((1, 128), lambda i: (0, 0)),
        out_shape=jax.ShapeDtypeStruct((1, 128), F32),
    )(edge_weight.reshape(e // 128, 128))
    minw = stats[0, 0].astype(jnp.bfloat16).astype(F32)
    maxw = stats[0, 1].astype(jnp.bfloat16).astype(F32)
    fw = (stats[0, 2] / e).astype(jnp.bfloat16).astype(F32)

    x = pl.pallas_call(
        _in_proj_body,
        grid=(nb,),
        in_specs=[
            pl.BlockSpec((NBLK, 128), lambda i: (i, 0)),
            pl.BlockSpec((128, 128), lambda i: (0, 0)),
            pl.BlockSpec((1, 128), lambda i: (0, 0)),
        ],
        out_specs=pl.BlockSpec((NBLK, 128), lambda i: (i, 0)),
        out_shape=jax.ShapeDtypeStruct((n, hid), F32),
    )(node_features, W_in, b_in.reshape(1, hid))

    edata = jnp.stack([src, dst,
                       lax.bitcast_convert_type(wq, I32)])

    sc_call = pl.kernel(
        _sc_edge_body,
        out_type=jax.ShapeDtypeStruct((2, 10240, 144), F32),
        mesh=plsc.VectorSubcoreMesh(core_axis_name="c", subcore_axis_name="s"),
        compiler_params=pltpu.CompilerParams(use_tc_tiling_on_sc=False,
                                             needs_layout_passes=False),
        scratch_types=[
            pltpu.VMEM((16,), F32),
            pltpu.VMEM((3, SUPC * K), I32),
            pltpu.VMEM((K,), I32),
            pltpu.VMEM((K, 8), F32),
            pltpu.VMEM((K, 8), F32),
            pltpu.VMEM((K, 8), F32),
            pltpu.VMEM((K, 8), F32),
            pltpu.VMEM((K, 128), F32),
            pltpu.VMEM((K, 128), F32),
            pltpu.VMEM((K, 144), F32),
            pltpu.VMEM_SHARED((10240, 144), F32),
            pltpu.SemaphoreType.DMA,
            pltpu.SemaphoreType.DMA,
        ],
    )

    for i in range(num_layers):
        xp, asd, mx = pl.pallas_call(
            functools.partial(_proj_body),
            grid=(nb,),
            in_specs=[
                pl.BlockSpec((NBLK, 128), lambda i: (i, 0)),
                pl.BlockSpec((128, 128), lambda i: (0, 0)),
                pl.BlockSpec((1, 128), lambda i: (0, 0)),
                pl.BlockSpec((1, 128), lambda i: (0, 0)),
            ],
            out_specs=[
                pl.BlockSpec((NBLK, 128), lambda i: (i, 0)),
                pl.BlockSpec((NBLK, 8), lambda i: (i, 0)),
                pl.BlockSpec((1, 8), lambda i: (0, 0)),
            ],
            out_shape=[
                jax.ShapeDtypeStruct((n, hid), F32),
                jax.ShapeDtypeStruct((n, 8), F32),
                jax.ShapeDtypeStruct((1, 8), F32),
            ],
        )(x, lin_w[i], att_src[i].reshape(1, hid), att_dst[i].reshape(1, hid))

        cei = ce[i]
        bound = mx[0, :4] + mx[0, 4:8] + jnp.maximum(
            jnp.maximum(cei * minw, cei * maxw), cei * fw)
        s = jnp.where(bound >= 0, bound, 0.2 * bound)
        consts = jnp.concatenate(
            [s, cei, jnp.stack([fw]), jnp.zeros((7,), F32)]).astype(F32)

        acc = sc_call(edata, asd, xp, consts)

        x = pl.pallas_call(
            functools.partial(_combine_body, with_res=(i > 0)),
            grid=(nb,),
            in_specs=[
                pl.BlockSpec((2, NBLK, 144), lambda i: (0, i, 0)),
                pl.BlockSpec((NBLK, 8), lambda i: (i, 0)),
                pl.BlockSpec((NBLK, 128), lambda i: (i, 0)),
                pl.BlockSpec((NBLK, 128), lambda i: (i, 0)),
                pl.BlockSpec((1, 128), lambda i: (0, 0)),
                pl.BlockSpec((1, 128), lambda i: (0, 0)),
                pl.BlockSpec((1, 128), lambda i: (0, 0)),
                pl.BlockSpec((1, 16), lambda i: (0, 0)),
            ],
            out_specs=pl.BlockSpec((NBLK, 128), lambda i: (i, 0)),
            out_shape=jax.ShapeDtypeStruct((n, hid), F32),
        )(acc, asd, xp, x, gat_bias[i].reshape(1, hid),
          ln_g[i].reshape(1, hid), ln_b[i].reshape(1, hid),
          consts.reshape(1, 16))

    g3, gstats = pl.pallas_call(
        _gate_body,
        grid=(nb,),
        in_specs=[
            pl.BlockSpec((NBLK, 128), lambda i: (i, 0)),
            pl.BlockSpec((128, 128), lambda i: (0, 0)),
            pl.BlockSpec((1, 128), lambda i: (0, 0)),
            pl.BlockSpec((1, 128), lambda i: (0, 0)),
            pl.BlockSpec((1, 1), lambda i: (0, 0)),
        ],
        out_specs=[
            pl.BlockSpec((NBLK, 1), lambda i: (i, 0)),
            pl.BlockSpec((1, 2), lambda i: (0, 0)),
        ],
        out_shape=[
            jax.ShapeDtypeStruct((n, 1), F32),
            jax.ShapeDtypeStruct((1, 2), F32),
        ],
        scratch_shapes=[pltpu.SMEM((1,), F32), pltpu.SMEM((1,), F32)],
    )(x, gate_w1, gate_b1.reshape(1, 128),
      gate_w2.astype(jnp.bfloat16).astype(F32),
      gate_b2.reshape(1, 1))

    out = pl.pallas_call(
        _pool_body,
        grid=(nb,),
        in_specs=[
            pl.BlockSpec((NBLK, 128), lambda i: (i, 0)),
            pl.BlockSpec((NBLK, 1), lambda i: (i, 0)),
            pl.BlockSpec((1, 1, NBLK), lambda i: (i, 0, 0)),
            pl.BlockSpec((1, 2), lambda i: (0, 0)),
            pl.BlockSpec((128, 128), lambda i: (0, 0)),
            pl.BlockSpec((1, 128), lambda i: (0, 0)),
            pl.BlockSpec((64, 128), lambda i: (0, 0)),
            pl.BlockSpec((1, 64), lambda i: (0, 0)),
            pl.BlockSpec((1, 64), lambda i: (0, 0)),
            pl.BlockSpec((1, 1), lambda i: (0, 0)),
        ],
        out_specs=pl.BlockSpec((1, 64), lambda i: (0, 0)),
        out_shape=jax.ShapeDtypeStruct((1, 64), F32),
        scratch_shapes=[pltpu.VMEM((64, 128), F32)],
    )(x, g3, batch.reshape(nb, 1, NBLK), gstats, mlp_w1,
      mlp_b1.reshape(1, 128), mlp_w2, mlp_b2.reshape(1, 64), mlp_w3,
      mlp_b3.reshape(1, 1))

    return out[0]
